# baseline (speedup 1.0000x reference)
import functools

import jax
import jax.numpy as jnp
from jax.experimental import pallas as pl

F32 = jnp.float32
BF16 = jnp.bfloat16


def _ln(x, g, b):
    m = jnp.mean(x, axis=-1, keepdims=True)
    m2 = jnp.mean(x * x, axis=-1, keepdims=True)
    v = jnp.maximum(m2 - m * m, 0.0)
    s = jax.lax.rsqrt(v + 1e-5)
    return (x - m) * s * g + b


def _enc_body(x_ref, w1_ref, b1_ref, g1_ref, be1_ref,
              w2_ref, b2_ref, g2_ref, be2_ref, h_ref):
    x = x_ref[...].astype(BF16)
    h1 = jnp.dot(x, w1_ref[...], preferred_element_type=F32)
    h1 = jax.nn.relu(_ln(h1 + b1_ref[...], g1_ref[...], be1_ref[...]))
    h2 = jnp.dot(h1.astype(BF16), w2_ref[...], preferred_element_type=F32)
    h2 = jax.nn.relu(_ln(h2 + b2_ref[...], g2_ref[...], be2_ref[...]))
    h_ref[...] = h2.astype(BF16)


def _route_body(h_ref, wr_ref, br_ref, gr_ref, ber_ref,
                wm_ref, bm_ref, wv_ref, bv_ref, eps_ref,
                probs_ref, mu_ref, lv_ref, zc_ref, *, E, L):
    bt = h_ref.shape[0]
    h = h_ref[...]
    logits = jnp.dot(h, wr_ref[...], preferred_element_type=F32) + br_ref[...]
    logits = _ln(logits, gr_ref[...], ber_ref[...])
    mx = jnp.max(logits, axis=-1, keepdims=True)
    ex = jnp.exp(logits - mx)
    probs = ex / jnp.sum(ex, axis=-1, keepdims=True)
    probs_ref[...] = probs

    mu = jnp.dot(h, wm_ref[...], preferred_element_type=F32) + bm_ref[...]
    lv = jnp.dot(h, wv_ref[...], preferred_element_type=F32) + bv_ref[...]
    mu_ref[...] = mu
    lv_ref[...] = lv

    v1 = jnp.max(probs, axis=-1, keepdims=True)
    i1 = jnp.argmax(probs, axis=-1)[:, None]
    lane = jax.lax.broadcasted_iota(jnp.int32, probs.shape, 1)
    oh1 = (lane == i1).astype(F32)
    masked = jnp.where(lane == i1, -jnp.inf, probs)
    v2 = jnp.max(masked, axis=-1, keepdims=True)
    i2 = jnp.argmax(masked, axis=-1)[:, None]
    oh2 = (lane == i2).astype(F32)

    wmu = v1 * oh1 + v2 * oh2
    muw = jnp.zeros((bt, L), F32)
    lv1 = jnp.zeros((bt, L), F32)
    lv2 = jnp.zeros((bt, L), F32)
    for e in range(E):
        msl = mu[:, e * L:(e + 1) * L]
        vsl = lv[:, e * L:(e + 1) * L]
        muw = muw + wmu[:, e:e + 1] * msl
        lv1 = lv1 + oh1[:, e:e + 1] * vsl
        lv2 = lv2 + oh2[:, e:e + 1] * vsl
    e1 = eps_ref[:, 0, :]
    e2 = eps_ref[:, 1, :]
    z = muw + v1 * e1 * jnp.exp(0.5 * lv1) + v2 * e2 * jnp.exp(0.5 * lv2)
    zc_ref[...] = z.astype(BF16)


def _dec_body(zc_ref, w1_ref, b1_ref, g1_ref, be1_ref,
              w2_ref, b2_ref, g2_ref, be2_ref, wo_ref, bo_ref, r_ref):
    z = zc_ref[...]
    d1 = jnp.dot(z, w1_ref[...], preferred_element_type=F32)
    d1 = jax.nn.relu(_ln(d1 + b1_ref[...], g1_ref[...], be1_ref[...]))
    d2 = jnp.dot(d1.astype(BF16), w2_ref[...], preferred_element_type=F32)
    d2 = jax.nn.relu(_ln(d2 + b2_ref[...], g2_ref[...], be2_ref[...]))
    r = jnp.dot(d2.astype(BF16), wo_ref[...], preferred_element_type=F32)
    r_ref[...] = r + bo_ref[...]


def _full(a):
    return pl.BlockSpec(a.shape, lambda i: (0,) * a.ndim)


def _row(v):
    return v.reshape(1, -1)


def kernel(x, params, eps):
    B, D = x.shape
    E = params["Wr"].shape[1]
    L = eps.shape[2]
    K = eps.shape[1]

    enc, dec = params["enc"], params["dec"]
    w1 = enc[0]["W"].astype(BF16)
    w2 = enc[1]["W"].astype(BF16)
    wr = params["Wr"].astype(BF16)
    wm = params["Wm"].astype(BF16)
    wv = params["Wv"].astype(BF16)
    wd1 = dec[0]["W"].astype(BF16)
    wd2 = dec[1]["W"].astype(BF16)
    wo = params["Wo"].astype(BF16)
    H = w2.shape[1]
    DO = wo.shape[1]

    bt1 = 512
    h = pl.pallas_call(
        _enc_body,
        grid=(B // bt1,),
        in_specs=[
            pl.BlockSpec((bt1, D), lambda i: (i, 0)),
            _full(w1), _full(_row(enc[0]["b"])), _full(_row(enc[0]["g"])), _full(_row(enc[0]["be"])),
            _full(w2), _full(_row(enc[1]["b"])), _full(_row(enc[1]["g"])), _full(_row(enc[1]["be"])),
        ],
        out_specs=pl.BlockSpec((bt1, H), lambda i: (i, 0)),
        out_shape=jax.ShapeDtypeStruct((B, H), BF16),
    )(x, w1, _row(enc[0]["b"]), _row(enc[0]["g"]), _row(enc[0]["be"]),
      w2, _row(enc[1]["b"]), _row(enc[1]["g"]), _row(enc[1]["be"]))

    bt2 = 256
    nchunks = 2
    BC = B // nchunks
    chunks = []
    for c in range(nchunks):
        hs = jax.lax.slice_in_dim(h, c * BC, (c + 1) * BC)
        es = jax.lax.slice_in_dim(eps, c * BC, (c + 1) * BC)
        out = pl.pallas_call(
            functools.partial(_route_body, E=E, L=L),
            grid=(BC // bt2,),
            in_specs=[
                pl.BlockSpec((bt2, H), lambda i: (i, 0)),
                _full(wr), _full(_row(params["br"])), _full(_row(params["gr"])), _full(_row(params["ber"])),
                _full(wm), _full(_row(params["bm"])),
                _full(wv), _full(_row(params["bv"])),
                pl.BlockSpec((bt2, K, L), lambda i: (i, 0, 0)),
            ],
            out_specs=[
                pl.BlockSpec((bt2, E), lambda i: (i, 0)),
                pl.BlockSpec((bt2, E * L), lambda i: (i, 0)),
                pl.BlockSpec((bt2, E * L), lambda i: (i, 0)),
                pl.BlockSpec((bt2, L), lambda i: (i, 0)),
            ],
            out_shape=[
                jax.ShapeDtypeStruct((BC, E), F32),
                jax.ShapeDtypeStruct((BC, E * L), F32),
                jax.ShapeDtypeStruct((BC, E * L), F32),
                jax.ShapeDtypeStruct((BC, L), BF16),
            ],
        )(hs, wr, _row(params["br"]), _row(params["gr"]), _row(params["ber"]),
          wm, _row(params["bm"]), wv, _row(params["bv"]), es)
        chunks.append(out)
    probs = jnp.concatenate([o[0] for o in chunks], axis=0)
    mu = jnp.concatenate([o[1].reshape(BC, E, L) for o in chunks], axis=0)
    lv = jnp.concatenate([o[2].reshape(BC, E, L) for o in chunks], axis=0)
    zc = jnp.concatenate([o[3] for o in chunks], axis=0)

    bt3 = 512
    recon = pl.pallas_call(
        _dec_body,
        grid=(B // bt3,),
        in_specs=[
            pl.BlockSpec((bt3, L), lambda i: (i, 0)),
            _full(wd1), _full(_row(dec[0]["b"])), _full(_row(dec[0]["g"])), _full(_row(dec[0]["be"])),
            _full(wd2), _full(_row(dec[1]["b"])), _full(_row(dec[1]["g"])), _full(_row(dec[1]["be"])),
            _full(wo), _full(_row(params["bo"])),
        ],
        out_specs=pl.BlockSpec((bt3, DO), lambda i: (i, 0)),
        out_shape=jax.ShapeDtypeStruct((B, DO), F32),
    )(zc, wd1, _row(dec[0]["b"]), _row(dec[0]["g"]), _row(dec[0]["be"]),
      wd2, _row(dec[1]["b"]), _row(dec[1]["g"]), _row(dec[1]["be"]),
      wo, _row(params["bo"]))

    return (recon, mu, lv, probs)


# speedup vs baseline: 1.3819x; 1.3819x over previous
import functools

import jax
import jax.numpy as jnp
from jax.experimental import pallas as pl
import jax.experimental.pallas.tpu as pltpu

F32 = jnp.float32
BF16 = jnp.bfloat16


def _ln(x, g, b):
    m = jnp.mean(x, axis=-1, keepdims=True)
    m2 = jnp.mean(x * x, axis=-1, keepdims=True)
    v = jnp.maximum(m2 - m * m, 0.0)
    s = jax.lax.rsqrt(v + 1e-5)
    return (x - m) * s * g + b


def _enc_body(x_ref, w1_ref, b1_ref, g1_ref, be1_ref,
              w2_ref, b2_ref, g2_ref, be2_ref, h_ref):
    x = x_ref[...].astype(BF16)
    h1 = jnp.dot(x, w1_ref[...], preferred_element_type=F32)
    h1 = jax.nn.relu(_ln(h1 + b1_ref[...], g1_ref[...], be1_ref[...]))
    h2 = jnp.dot(h1.astype(BF16), w2_ref[...], preferred_element_type=F32)
    h2 = jax.nn.relu(_ln(h2 + b2_ref[...], g2_ref[...], be2_ref[...]))
    h_ref[...] = h2.astype(BF16)


def _route_body(h_ref, wr_ref, br_ref, gr_ref, ber_ref,
                wm_ref, bm_ref, wv_ref, bv_ref, eps_ref,
                probs_ref, mu_ref, lv_ref, zc_ref,
                mu_s, lv_s, sems, *, E, L, bt):
    i = pl.program_id(0)
    nb = pl.num_programs(0)

    def copies(blk):
        out = []
        for t_idx, (s_ref, o_ref) in enumerate(((mu_s, mu_ref), (lv_s, lv_ref))):
            for e in range(E):
                out.append(pltpu.make_async_copy(
                    s_ref.at[:, e * L:(e + 1) * L],
                    o_ref.at[pl.ds(blk * bt, bt), e, :],
                    sems.at[t_idx, e],
                ))
        return out

    @pl.when(i > 0)
    def _():
        for c in copies(i - 1):
            c.wait()

    h = h_ref[...]
    logits = jnp.dot(h, wr_ref[...], preferred_element_type=F32) + br_ref[...]
    logits = _ln(logits, gr_ref[...], ber_ref[...])
    mx = jnp.max(logits, axis=-1, keepdims=True)
    ex = jnp.exp(logits - mx)
    probs = ex / jnp.sum(ex, axis=-1, keepdims=True)
    probs_ref[...] = probs

    mu = jnp.dot(h, wm_ref[...], preferred_element_type=F32) + bm_ref[...]
    lv = jnp.dot(h, wv_ref[...], preferred_element_type=F32) + bv_ref[...]
    mu_s[...] = mu
    lv_s[...] = lv

    for c in copies(i):
        c.start()

    v1 = jnp.max(probs, axis=-1, keepdims=True)
    i1 = jnp.argmax(probs, axis=-1)[:, None]
    lane = jax.lax.broadcasted_iota(jnp.int32, probs.shape, 1)
    oh1 = (lane == i1).astype(F32)
    masked = jnp.where(lane == i1, -jnp.inf, probs)
    v2 = jnp.max(masked, axis=-1, keepdims=True)
    i2 = jnp.argmax(masked, axis=-1)[:, None]
    oh2 = (lane == i2).astype(F32)

    wmu = v1 * oh1 + v2 * oh2
    muw = jnp.zeros((bt, L), F32)
    lv1 = jnp.zeros((bt, L), F32)
    lv2 = jnp.zeros((bt, L), F32)
    for e in range(E):
        msl = mu[:, e * L:(e + 1) * L]
        vsl = lv[:, e * L:(e + 1) * L]
        muw = muw + wmu[:, e:e + 1] * msl
        lv1 = lv1 + oh1[:, e:e + 1] * vsl
        lv2 = lv2 + oh2[:, e:e + 1] * vsl
    e1 = eps_ref[:, 0, :]
    e2 = eps_ref[:, 1, :]
    z = muw + v1 * e1 * jnp.exp(0.5 * lv1) + v2 * e2 * jnp.exp(0.5 * lv2)
    zc_ref[...] = z.astype(BF16)

    @pl.when(i == nb - 1)
    def _():
        for c in copies(i):
            c.wait()


def _dec_body(zc_ref, w1_ref, b1_ref, g1_ref, be1_ref,
              w2_ref, b2_ref, g2_ref, be2_ref, wo_ref, bo_ref, r_ref):
    z = zc_ref[...]
    d1 = jnp.dot(z, w1_ref[...], preferred_element_type=F32)
    d1 = jax.nn.relu(_ln(d1 + b1_ref[...], g1_ref[...], be1_ref[...]))
    d2 = jnp.dot(d1.astype(BF16), w2_ref[...], preferred_element_type=F32)
    d2 = jax.nn.relu(_ln(d2 + b2_ref[...], g2_ref[...], be2_ref[...]))
    r = jnp.dot(d2.astype(BF16), wo_ref[...], preferred_element_type=F32)
    r_ref[...] = r + bo_ref[...]


def _full(a):
    return pl.BlockSpec(a.shape, lambda i: (0,) * a.ndim)


def _row(v):
    return v.reshape(1, -1)


def kernel(x, params, eps):
    B, D = x.shape
    E = params["Wr"].shape[1]
    L = eps.shape[2]
    K = eps.shape[1]

    enc, dec = params["enc"], params["dec"]
    w1 = enc[0]["W"].astype(BF16)
    w2 = enc[1]["W"].astype(BF16)
    wr = params["Wr"].astype(BF16)
    wm = params["Wm"].astype(BF16)
    wv = params["Wv"].astype(BF16)
    wd1 = dec[0]["W"].astype(BF16)
    wd2 = dec[1]["W"].astype(BF16)
    wo = params["Wo"].astype(BF16)
    H = w2.shape[1]
    DO = wo.shape[1]

    bt1 = 512
    h = pl.pallas_call(
        _enc_body,
        grid=(B // bt1,),
        in_specs=[
            pl.BlockSpec((bt1, D), lambda i: (i, 0)),
            _full(w1), _full(_row(enc[0]["b"])), _full(_row(enc[0]["g"])), _full(_row(enc[0]["be"])),
            _full(w2), _full(_row(enc[1]["b"])), _full(_row(enc[1]["g"])), _full(_row(enc[1]["be"])),
        ],
        out_specs=pl.BlockSpec((bt1, H), lambda i: (i, 0)),
        out_shape=jax.ShapeDtypeStruct((B, H), BF16),
    )(x, w1, _row(enc[0]["b"]), _row(enc[0]["g"]), _row(enc[0]["be"]),
      w2, _row(enc[1]["b"]), _row(enc[1]["g"]), _row(enc[1]["be"]))

    bt2 = 256
    probs, mu, lv, zc = pl.pallas_call(
        functools.partial(_route_body, E=E, L=L, bt=bt2),
        grid=(B // bt2,),
        in_specs=[
            pl.BlockSpec((bt2, H), lambda i: (i, 0)),
            _full(wr), _full(_row(params["br"])), _full(_row(params["gr"])), _full(_row(params["ber"])),
            _full(wm), _full(_row(params["bm"])),
            _full(wv), _full(_row(params["bv"])),
            pl.BlockSpec((bt2, K, L), lambda i: (i, 0, 0)),
        ],
        out_specs=[
            pl.BlockSpec((bt2, E), lambda i: (i, 0)),
            pl.BlockSpec(memory_space=pl.ANY),
            pl.BlockSpec(memory_space=pl.ANY),
            pl.BlockSpec((bt2, L), lambda i: (i, 0)),
        ],
        out_shape=[
            jax.ShapeDtypeStruct((B, E), F32),
            jax.ShapeDtypeStruct((B, E, L), F32),
            jax.ShapeDtypeStruct((B, E, L), F32),
            jax.ShapeDtypeStruct((B, L), BF16),
        ],
        scratch_shapes=[
            pltpu.VMEM((bt2, E * L), F32),
            pltpu.VMEM((bt2, E * L), F32),
            pltpu.SemaphoreType.DMA((2, E)),
        ],
    )(h, wr, _row(params["br"]), _row(params["gr"]), _row(params["ber"]),
      wm, _row(params["bm"]), wv, _row(params["bv"]), eps)

    bt3 = 512
    recon = pl.pallas_call(
        _dec_body,
        grid=(B // bt3,),
        in_specs=[
            pl.BlockSpec((bt3, L), lambda i: (i, 0)),
            _full(wd1), _full(_row(dec[0]["b"])), _full(_row(dec[0]["g"])), _full(_row(dec[0]["be"])),
            _full(wd2), _full(_row(dec[1]["b"])), _full(_row(dec[1]["g"])), _full(_row(dec[1]["be"])),
            _full(wo), _full(_row(params["bo"])),
        ],
        out_specs=pl.BlockSpec((bt3, DO), lambda i: (i, 0)),
        out_shape=jax.ShapeDtypeStruct((B, DO), F32),
    )(zc, wd1, _row(dec[0]["b"]), _row(dec[0]["g"]), _row(dec[0]["be"]),
      wd2, _row(dec[1]["b"]), _row(dec[1]["g"]), _row(dec[1]["be"]),
      wo, _row(params["bo"]))

    return (recon, mu, lv, probs)


# speedup vs baseline: 1.4900x; 1.0783x over previous
import functools

import jax
import jax.numpy as jnp
from jax.experimental import pallas as pl
import jax.experimental.pallas.tpu as pltpu

F32 = jnp.float32
BF16 = jnp.bfloat16


def _ln(x, g, b):
    m = jnp.mean(x, axis=-1, keepdims=True)
    m2 = jnp.mean(x * x, axis=-1, keepdims=True)
    v = jnp.maximum(m2 - m * m, 0.0)
    s = jax.lax.rsqrt(v + 1e-5)
    return (x - m) * s * g + b


def _enc_body(x_ref, w1_ref, b1_ref, g1_ref, be1_ref,
              w2_ref, b2_ref, g2_ref, be2_ref, h_ref):
    x = x_ref[...].astype(BF16)
    h1 = jnp.dot(x, w1_ref[...], preferred_element_type=F32)
    h1 = jax.nn.relu(_ln(h1 + b1_ref[...], g1_ref[...], be1_ref[...]))
    h2 = jnp.dot(h1.astype(BF16), w2_ref[...], preferred_element_type=F32)
    h2 = jax.nn.relu(_ln(h2 + b2_ref[...], g2_ref[...], be2_ref[...]))
    h_ref[...] = h2.astype(BF16)


def _route_body(h_ref, wr_ref, br_ref, gr_ref, ber_ref,
                wm_ref, bm_ref, wv_ref, bv_ref, eps_ref,
                probs_ref, mu_ref, lv_ref, zc_ref,
                mu_s0, lv_s0, mu_s1, lv_s1, sems, *, E, L, bt):
    i = pl.program_id(0)
    nb = pl.num_programs(0)
    slots = ((mu_s0, lv_s0), (mu_s1, lv_s1))

    def copies(slot, blk):
        mu_s, lv_s = slots[slot]
        out = []
        for t_idx, (s_ref, o_ref) in enumerate(((mu_s, mu_ref), (lv_s, lv_ref))):
            for e in range(E):
                out.append(pltpu.make_async_copy(
                    s_ref.at[:, e * L:(e + 1) * L],
                    o_ref.at[pl.ds(blk * bt, bt), e, :],
                    sems.at[slot, t_idx, e],
                ))
        return out

    @pl.when((i > 1) & (i % 2 == 0))
    def _():
        for c in copies(0, i - 2):
            c.wait()

    @pl.when((i > 1) & (i % 2 == 1))
    def _():
        for c in copies(1, i - 2):
            c.wait()

    h = h_ref[...]
    logits = jnp.dot(h, wr_ref[...], preferred_element_type=F32) + br_ref[...]
    logits = _ln(logits, gr_ref[...], ber_ref[...])
    mx = jnp.max(logits, axis=-1, keepdims=True)
    ex = jnp.exp(logits - mx)
    probs = ex / jnp.sum(ex, axis=-1, keepdims=True)
    probs_ref[...] = probs

    mu = jnp.dot(h, wm_ref[...], preferred_element_type=F32) + bm_ref[...]
    lv = jnp.dot(h, wv_ref[...], preferred_element_type=F32) + bv_ref[...]

    @pl.when(i % 2 == 0)
    def _():
        mu_s0[...] = mu
        lv_s0[...] = lv
        for c in copies(0, i):
            c.start()

    @pl.when(i % 2 == 1)
    def _():
        mu_s1[...] = mu
        lv_s1[...] = lv
        for c in copies(1, i):
            c.start()

    v1 = jnp.max(probs, axis=-1, keepdims=True)
    i1 = jnp.argmax(probs, axis=-1)[:, None]
    lane = jax.lax.broadcasted_iota(jnp.int32, probs.shape, 1)
    oh1 = (lane == i1).astype(F32)
    masked = jnp.where(lane == i1, -jnp.inf, probs)
    v2 = jnp.max(masked, axis=-1, keepdims=True)
    i2 = jnp.argmax(masked, axis=-1)[:, None]
    oh2 = (lane == i2).astype(F32)

    wmu = v1 * oh1 + v2 * oh2
    muw = jnp.zeros((bt, L), F32)
    lv1 = jnp.zeros((bt, L), F32)
    lv2 = jnp.zeros((bt, L), F32)
    for e in range(E):
        msl = mu[:, e * L:(e + 1) * L]
        vsl = lv[:, e * L:(e + 1) * L]
        muw = muw + wmu[:, e:e + 1] * msl
        lv1 = lv1 + oh1[:, e:e + 1] * vsl
        lv2 = lv2 + oh2[:, e:e + 1] * vsl
    e1 = eps_ref[:, 0, :]
    e2 = eps_ref[:, 1, :]
    z = muw + v1 * e1 * jnp.exp(0.5 * lv1) + v2 * e2 * jnp.exp(0.5 * lv2)
    zc_ref[...] = z.astype(BF16)

    @pl.when((i == nb - 1) & (i % 2 == 0))
    def _():
        for c in copies(1, i - 1):
            c.wait()
        for c in copies(0, i):
            c.wait()

    @pl.when((i == nb - 1) & (i % 2 == 1))
    def _():
        for c in copies(0, i - 1):
            c.wait()
        for c in copies(1, i):
            c.wait()


def _dec_body(zc_ref, w1_ref, b1_ref, g1_ref, be1_ref,
              w2_ref, b2_ref, g2_ref, be2_ref, wo_ref, bo_ref, r_ref):
    z = zc_ref[...]
    d1 = jnp.dot(z, w1_ref[...], preferred_element_type=F32)
    d1 = jax.nn.relu(_ln(d1 + b1_ref[...], g1_ref[...], be1_ref[...]))
    d2 = jnp.dot(d1.astype(BF16), w2_ref[...], preferred_element_type=F32)
    d2 = jax.nn.relu(_ln(d2 + b2_ref[...], g2_ref[...], be2_ref[...]))
    r = jnp.dot(d2.astype(BF16), wo_ref[...], preferred_element_type=F32)
    r_ref[...] = r + bo_ref[...]


def _full(a):
    return pl.BlockSpec(a.shape, lambda i: (0,) * a.ndim)


def _row(v):
    return v.reshape(1, -1)


def kernel(x, params, eps):
    B, D = x.shape
    E = params["Wr"].shape[1]
    L = eps.shape[2]
    K = eps.shape[1]

    enc, dec = params["enc"], params["dec"]
    w1 = enc[0]["W"].astype(BF16)
    w2 = enc[1]["W"].astype(BF16)
    wr = params["Wr"].astype(BF16)
    wm = params["Wm"].astype(BF16)
    wv = params["Wv"].astype(BF16)
    wd1 = dec[0]["W"].astype(BF16)
    wd2 = dec[1]["W"].astype(BF16)
    wo = params["Wo"].astype(BF16)
    H = w2.shape[1]
    DO = wo.shape[1]

    bt1 = 512
    h = pl.pallas_call(
        _enc_body,
        grid=(B // bt1,),
        in_specs=[
            pl.BlockSpec((bt1, D), lambda i: (i, 0)),
            _full(w1), _full(_row(enc[0]["b"])), _full(_row(enc[0]["g"])), _full(_row(enc[0]["be"])),
            _full(w2), _full(_row(enc[1]["b"])), _full(_row(enc[1]["g"])), _full(_row(enc[1]["be"])),
        ],
        out_specs=pl.BlockSpec((bt1, H), lambda i: (i, 0)),
        out_shape=jax.ShapeDtypeStruct((B, H), BF16),
    )(x, w1, _row(enc[0]["b"]), _row(enc[0]["g"]), _row(enc[0]["be"]),
      w2, _row(enc[1]["b"]), _row(enc[1]["g"]), _row(enc[1]["be"]))

    bt2 = 256
    probs, mu, lv, zc = pl.pallas_call(
        functools.partial(_route_body, E=E, L=L, bt=bt2),
        grid=(B // bt2,),
        in_specs=[
            pl.BlockSpec((bt2, H), lambda i: (i, 0)),
            _full(wr), _full(_row(params["br"])), _full(_row(params["gr"])), _full(_row(params["ber"])),
            _full(wm), _full(_row(params["bm"])),
            _full(wv), _full(_row(params["bv"])),
            pl.BlockSpec((bt2, K, L), lambda i: (i, 0, 0)),
        ],
        out_specs=[
            pl.BlockSpec((bt2, E), lambda i: (i, 0)),
            pl.BlockSpec(memory_space=pl.ANY),
            pl.BlockSpec(memory_space=pl.ANY),
            pl.BlockSpec((bt2, L), lambda i: (i, 0)),
        ],
        out_shape=[
            jax.ShapeDtypeStruct((B, E), F32),
            jax.ShapeDtypeStruct((B, E, L), F32),
            jax.ShapeDtypeStruct((B, E, L), F32),
            jax.ShapeDtypeStruct((B, L), BF16),
        ],
        scratch_shapes=[
            pltpu.VMEM((bt2, E * L), F32),
            pltpu.VMEM((bt2, E * L), F32),
            pltpu.VMEM((bt2, E * L), F32),
            pltpu.VMEM((bt2, E * L), F32),
            pltpu.SemaphoreType.DMA((2, 2, E)),
        ],
    )(h, wr, _row(params["br"]), _row(params["gr"]), _row(params["ber"]),
      wm, _row(params["bm"]), wv, _row(params["bv"]), eps)

    bt3 = 512
    recon = pl.pallas_call(
        _dec_body,
        grid=(B // bt3,),
        in_specs=[
            pl.BlockSpec((bt3, L), lambda i: (i, 0)),
            _full(wd1), _full(_row(dec[0]["b"])), _full(_row(dec[0]["g"])), _full(_row(dec[0]["be"])),
            _full(wd2), _full(_row(dec[1]["b"])), _full(_row(dec[1]["g"])), _full(_row(dec[1]["be"])),
            _full(wo), _full(_row(params["bo"])),
        ],
        out_specs=pl.BlockSpec((bt3, DO), lambda i: (i, 0)),
        out_shape=jax.ShapeDtypeStruct((B, DO), F32),
    )(zc, wd1, _row(dec[0]["b"]), _row(dec[0]["g"]), _row(dec[0]["be"]),
      wd2, _row(dec[1]["b"]), _row(dec[1]["g"]), _row(dec[1]["be"]),
      wo, _row(params["bo"]))

    return (recon, mu, lv, probs)


# speedup vs baseline: 1.5033x; 1.0089x over previous
import functools

import jax
import jax.numpy as jnp
from jax.experimental import pallas as pl
import jax.experimental.pallas.tpu as pltpu

F32 = jnp.float32
BF16 = jnp.bfloat16


def _ln(x, g, b):
    m = jnp.mean(x, axis=-1, keepdims=True)
    m2 = jnp.mean(x * x, axis=-1, keepdims=True)
    v = jnp.maximum(m2 - m * m, 0.0)
    s = jax.lax.rsqrt(v + 1e-5)
    return (x - m) * s * g + b


def _enc_body(x_ref, w1_ref, b1_ref, g1_ref, be1_ref,
              w2_ref, b2_ref, g2_ref, be2_ref, h_ref):
    x = x_ref[...].astype(BF16)
    h1 = jnp.dot(x, w1_ref[...], preferred_element_type=F32)
    h1 = jax.nn.relu(_ln(h1 + b1_ref[...], g1_ref[...], be1_ref[...]))
    h2 = jnp.dot(h1.astype(BF16), w2_ref[...], preferred_element_type=F32)
    h2 = jax.nn.relu(_ln(h2 + b2_ref[...], g2_ref[...], be2_ref[...]))
    h_ref[...] = h2.astype(BF16)


def _route_body(h_ref, wr_ref, br_ref, gr_ref, ber_ref,
                wm_ref, bm_ref, wv_ref, bv_ref, eps_ref,
                probs_ref, mu_ref, lv_ref, zc_ref, *, E, L, bt):
    h = h_ref[...]
    logits = jnp.dot(h, wr_ref[...], preferred_element_type=F32) + br_ref[...]
    logits = _ln(logits, gr_ref[...], ber_ref[...])
    mx = jnp.max(logits, axis=-1, keepdims=True)
    ex = jnp.exp(logits - mx)
    probs = ex / jnp.sum(ex, axis=-1, keepdims=True)
    probs_ref[...] = probs

    mu = jnp.dot(h, wm_ref[...], preferred_element_type=F32) + bm_ref[...]
    lv = jnp.dot(h, wv_ref[...], preferred_element_type=F32) + bv_ref[...]
    mu_ref[...] = mu.reshape(bt, E, L)
    lv_ref[...] = lv.reshape(bt, E, L)

    v1 = jnp.max(probs, axis=-1, keepdims=True)
    i1 = jnp.argmax(probs, axis=-1)[:, None]
    lane = jax.lax.broadcasted_iota(jnp.int32, probs.shape, 1)
    oh1 = (lane == i1).astype(F32)
    masked = jnp.where(lane == i1, -jnp.inf, probs)
    v2 = jnp.max(masked, axis=-1, keepdims=True)
    i2 = jnp.argmax(masked, axis=-1)[:, None]
    oh2 = (lane == i2).astype(F32)

    wmu = v1 * oh1 + v2 * oh2
    muw = jnp.zeros((bt, L), F32)
    lv1 = jnp.zeros((bt, L), F32)
    lv2 = jnp.zeros((bt, L), F32)
    for e in range(E):
        msl = mu[:, e * L:(e + 1) * L]
        vsl = lv[:, e * L:(e + 1) * L]
        muw = muw + wmu[:, e:e + 1] * msl
        lv1 = lv1 + oh1[:, e:e + 1] * vsl
        lv2 = lv2 + oh2[:, e:e + 1] * vsl
    e1 = eps_ref[:, 0, :]
    e2 = eps_ref[:, 1, :]
    z = muw + v1 * e1 * jnp.exp(0.5 * lv1) + v2 * e2 * jnp.exp(0.5 * lv2)
    zc_ref[...] = z.astype(BF16)


def _dec_body(zc_ref, w1_ref, b1_ref, g1_ref, be1_ref,
              w2_ref, b2_ref, g2_ref, be2_ref, wo_ref, bo_ref, r_ref):
    z = zc_ref[...]
    d1 = jnp.dot(z, w1_ref[...], preferred_element_type=F32)
    d1 = jax.nn.relu(_ln(d1 + b1_ref[...], g1_ref[...], be1_ref[...]))
    d2 = jnp.dot(d1.astype(BF16), w2_ref[...], preferred_element_type=F32)
    d2 = jax.nn.relu(_ln(d2 + b2_ref[...], g2_ref[...], be2_ref[...]))
    r = jnp.dot(d2.astype(BF16), wo_ref[...], preferred_element_type=F32)
    r_ref[...] = r + bo_ref[...]


def _full(a):
    return pl.BlockSpec(a.shape, lambda i: (0,) * a.ndim)


def _row(v):
    return v.reshape(1, -1)


def kernel(x, params, eps):
    B, D = x.shape
    E = params["Wr"].shape[1]
    L = eps.shape[2]
    K = eps.shape[1]

    enc, dec = params["enc"], params["dec"]
    w1 = enc[0]["W"].astype(BF16)
    w2 = enc[1]["W"].astype(BF16)
    wr = params["Wr"].astype(BF16)
    wm = params["Wm"].astype(BF16)
    wv = params["Wv"].astype(BF16)
    wd1 = dec[0]["W"].astype(BF16)
    wd2 = dec[1]["W"].astype(BF16)
    wo = params["Wo"].astype(BF16)
    H = w2.shape[1]
    DO = wo.shape[1]

    bt1 = 1024
    h = pl.pallas_call(
        _enc_body,
        grid=(B // bt1,),
        in_specs=[
            pl.BlockSpec((bt1, D), lambda i: (i, 0)),
            _full(w1), _full(_row(enc[0]["b"])), _full(_row(enc[0]["g"])), _full(_row(enc[0]["be"])),
            _full(w2), _full(_row(enc[1]["b"])), _full(_row(enc[1]["g"])), _full(_row(enc[1]["be"])),
        ],
        out_specs=pl.BlockSpec((bt1, H), lambda i: (i, 0)),
        out_shape=jax.ShapeDtypeStruct((B, H), BF16),
    )(x, w1, _row(enc[0]["b"]), _row(enc[0]["g"]), _row(enc[0]["be"]),
      w2, _row(enc[1]["b"]), _row(enc[1]["g"]), _row(enc[1]["be"]))

    bt2 = 256
    probs, mu, lv, zc = pl.pallas_call(
        functools.partial(_route_body, E=E, L=L, bt=bt2),
        grid=(B // bt2,),
        in_specs=[
            pl.BlockSpec((bt2, H), lambda i: (i, 0)),
            _full(wr), _full(_row(params["br"])), _full(_row(params["gr"])), _full(_row(params["ber"])),
            _full(wm), _full(_row(params["bm"])),
            _full(wv), _full(_row(params["bv"])),
            pl.BlockSpec((bt2, K, L), lambda i: (i, 0, 0)),
        ],
        out_specs=[
            pl.BlockSpec((bt2, E), lambda i: (i, 0)),
            pl.BlockSpec((bt2, E, L), lambda i: (i, 0, 0)),
            pl.BlockSpec((bt2, E, L), lambda i: (i, 0, 0)),
            pl.BlockSpec((bt2, L), lambda i: (i, 0)),
        ],
        out_shape=[
            jax.ShapeDtypeStruct((B, E), F32),
            jax.ShapeDtypeStruct((B, E, L), F32),
            jax.ShapeDtypeStruct((B, E, L), F32),
            jax.ShapeDtypeStruct((B, L), BF16),
        ],
    )(h, wr, _row(params["br"]), _row(params["gr"]), _row(params["ber"]),
      wm, _row(params["bm"]), wv, _row(params["bv"]), eps)

    bt3 = 1024
    recon = pl.pallas_call(
        _dec_body,
        grid=(B // bt3,),
        in_specs=[
            pl.BlockSpec((bt3, L), lambda i: (i, 0)),
            _full(wd1), _full(_row(dec[0]["b"])), _full(_row(dec[0]["g"])), _full(_row(dec[0]["be"])),
            _full(wd2), _full(_row(dec[1]["b"])), _full(_row(dec[1]["g"])), _full(_row(dec[1]["be"])),
            _full(wo), _full(_row(params["bo"])),
        ],
        out_specs=pl.BlockSpec((bt3, DO), lambda i: (i, 0)),
        out_shape=jax.ShapeDtypeStruct((B, DO), F32),
    )(zc, wd1, _row(dec[0]["b"]), _row(dec[0]["g"]), _row(dec[0]["be"]),
      wd2, _row(dec[1]["b"]), _row(dec[1]["g"]), _row(dec[1]["be"]),
      wo, _row(params["bo"]))

    return (recon, mu, lv, probs)


# speedup vs baseline: 1.5108x; 1.0050x over previous
import functools

import jax
import jax.numpy as jnp
from jax.experimental import pallas as pl

F32 = jnp.float32
BF16 = jnp.bfloat16


def _ln(x, g, b):
    m = jnp.mean(x, axis=-1, keepdims=True)
    m2 = jnp.mean(x * x, axis=-1, keepdims=True)
    v = jnp.maximum(m2 - m * m, 0.0)
    s = jax.lax.rsqrt(v + 1e-5)
    return (x - m) * s * g + b


def _enc_body(x_ref, w1_ref, b1_ref, g1_ref, be1_ref,
              w2_ref, b2_ref, g2_ref, be2_ref, h_ref):
    x = x_ref[...].astype(BF16)
    h1 = jnp.dot(x, w1_ref[...], preferred_element_type=F32)
    h1 = jax.nn.relu(_ln(h1 + b1_ref[...], g1_ref[...], be1_ref[...]))
    h2 = jnp.dot(h1.astype(BF16), w2_ref[...], preferred_element_type=F32)
    h2 = jax.nn.relu(_ln(h2 + b2_ref[...], g2_ref[...], be2_ref[...]))
    h_ref[...] = h2.astype(BF16)


def _route_body(h_ref, wr_ref, br_ref, gr_ref, ber_ref,
                wm_ref, bm_ref, wv_ref, bv_ref, eps_ref,
                probs_ref, mu_ref, lv_ref, zc_ref, *, E, L, bt):
    h = h_ref[...]
    logits = jnp.dot(h, wr_ref[...], preferred_element_type=F32) + br_ref[...]
    logits = _ln(logits, gr_ref[...], ber_ref[...])
    mx = jnp.max(logits, axis=-1, keepdims=True)
    ex = jnp.exp(logits - mx)
    probs = ex / jnp.sum(ex, axis=-1, keepdims=True)
    probs_ref[...] = probs

    mu = jnp.dot(h, wm_ref[...], preferred_element_type=F32) + bm_ref[...]
    lv = jnp.dot(h, wv_ref[...], preferred_element_type=F32) + bv_ref[...]
    mu_ref[...] = mu.reshape(bt, E, L)
    lv_ref[...] = lv.reshape(bt, E, L)

    v1 = jnp.max(probs, axis=-1, keepdims=True)
    i1 = jnp.argmax(probs, axis=-1)[:, None]
    lane = jax.lax.broadcasted_iota(jnp.int32, probs.shape, 1)
    oh1 = (lane == i1).astype(F32)
    masked = jnp.where(lane == i1, -jnp.inf, probs)
    v2 = jnp.max(masked, axis=-1, keepdims=True)
    i2 = jnp.argmax(masked, axis=-1)[:, None]
    oh2 = (lane == i2).astype(F32)

    wmu = v1 * oh1 + v2 * oh2
    muw = jnp.zeros((bt, L), F32)
    lv1 = jnp.zeros((bt, L), F32)
    lv2 = jnp.zeros((bt, L), F32)
    for e in range(E):
        msl = mu[:, e * L:(e + 1) * L]
        vsl = lv[:, e * L:(e + 1) * L]
        muw = muw + wmu[:, e:e + 1] * msl
        lv1 = lv1 + oh1[:, e:e + 1] * vsl
        lv2 = lv2 + oh2[:, e:e + 1] * vsl
    e1 = eps_ref[:, 0, :]
    e2 = eps_ref[:, 1, :]
    z = muw + v1 * e1 * jnp.exp(0.5 * lv1) + v2 * e2 * jnp.exp(0.5 * lv2)
    zc_ref[...] = z.astype(BF16)


def _dec_body(zc_ref, w1_ref, b1_ref, g1_ref, be1_ref,
              w2_ref, b2_ref, g2_ref, be2_ref, wo_ref, bo_ref, r_ref):
    z = zc_ref[...]
    d1 = jnp.dot(z, w1_ref[...], preferred_element_type=F32)
    d1 = jax.nn.relu(_ln(d1 + b1_ref[...], g1_ref[...], be1_ref[...]))
    d2 = jnp.dot(d1.astype(BF16), w2_ref[...], preferred_element_type=F32)
    d2 = jax.nn.relu(_ln(d2 + b2_ref[...], g2_ref[...], be2_ref[...]))
    r = jnp.dot(d2.astype(BF16), wo_ref[...], preferred_element_type=F32)
    r_ref[...] = r + bo_ref[...]


def _full(a):
    return pl.BlockSpec(a.shape, lambda i: (0,) * a.ndim)


def _row(v):
    return v.reshape(1, -1)


def kernel(x, params, eps):
    B, D = x.shape
    E = params["Wr"].shape[1]
    L = eps.shape[2]
    K = eps.shape[1]

    enc, dec = params["enc"], params["dec"]
    w1 = enc[0]["W"].astype(BF16)
    w2 = enc[1]["W"].astype(BF16)
    wr = params["Wr"].astype(BF16)
    wm = params["Wm"].astype(BF16)
    wv = params["Wv"].astype(BF16)
    wd1 = dec[0]["W"].astype(BF16)
    wd2 = dec[1]["W"].astype(BF16)
    wo = params["Wo"].astype(BF16)
    H = w2.shape[1]
    DO = wo.shape[1]

    bt1 = 1024
    h = pl.pallas_call(
        _enc_body,
        grid=(B // bt1,),
        in_specs=[
            pl.BlockSpec((bt1, D), lambda i: (i, 0)),
            _full(w1), _full(_row(enc[0]["b"])), _full(_row(enc[0]["g"])), _full(_row(enc[0]["be"])),
            _full(w2), _full(_row(enc[1]["b"])), _full(_row(enc[1]["g"])), _full(_row(enc[1]["be"])),
        ],
        out_specs=pl.BlockSpec((bt1, H), lambda i: (i, 0)),
        out_shape=jax.ShapeDtypeStruct((B, H), BF16),
    )(x, w1, _row(enc[0]["b"]), _row(enc[0]["g"]), _row(enc[0]["be"]),
      w2, _row(enc[1]["b"]), _row(enc[1]["g"]), _row(enc[1]["be"]))

    bt2 = 256
    probs, mu, lv, zc = pl.pallas_call(
        functools.partial(_route_body, E=E, L=L, bt=bt2),
        grid=(B // bt2,),
        in_specs=[
            pl.BlockSpec((bt2, H), lambda i: (i, 0)),
            _full(wr), _full(_row(params["br"])), _full(_row(params["gr"])), _full(_row(params["ber"])),
            _full(wm), _full(_row(params["bm"])),
            _full(wv), _full(_row(params["bv"])),
            pl.BlockSpec((bt2, K, L), lambda i: (i, 0, 0)),
        ],
        out_specs=[
            pl.BlockSpec((bt2, E), lambda i: (i, 0)),
            pl.BlockSpec((bt2, E, L), lambda i: (i, 0, 0)),
            pl.BlockSpec((bt2, E, L), lambda i: (i, 0, 0)),
            pl.BlockSpec((bt2, L), lambda i: (i, 0)),
        ],
        out_shape=[
            jax.ShapeDtypeStruct((B, E), F32),
            jax.ShapeDtypeStruct((B, E, L), F32),
            jax.ShapeDtypeStruct((B, E, L), F32),
            jax.ShapeDtypeStruct((B, L), BF16),
        ],
    )(h, wr, _row(params["br"]), _row(params["gr"]), _row(params["ber"]),
      wm, _row(params["bm"]), wv, _row(params["bv"]), eps)

    bt3 = 1024
    recon = pl.pallas_call(
        _dec_body,
        grid=(B // bt3,),
        in_specs=[
            pl.BlockSpec((bt3, L), lambda i: (i, 0)),
            _full(wd1), _full(_row(dec[0]["b"])), _full(_row(dec[0]["g"])), _full(_row(dec[0]["be"])),
            _full(wd2), _full(_row(dec[1]["b"])), _full(_row(dec[1]["g"])), _full(_row(dec[1]["be"])),
            _full(wo), _full(_row(params["bo"])),
        ],
        out_specs=pl.BlockSpec((bt3, DO), lambda i: (i, 0)),
        out_shape=jax.ShapeDtypeStruct((B, DO), F32),
    )(zc, wd1, _row(dec[0]["b"]), _row(dec[0]["g"]), _row(dec[0]["be"]),
      wd2, _row(dec[1]["b"]), _row(dec[1]["g"]), _row(dec[1]["be"]),
      wo, _row(params["bo"]))

    return (recon, mu, lv, probs)
